# baseline (speedup 1.0000x reference)
import jax
import jax.numpy as jnp
from jax.experimental import pallas as pl
from jax.experimental.pallas import tpu as pltpu

_N = 10000
_BM = 200
_NM = _N // _BM
_D = 128

_LANCZOS = (
    676.5203681218851, -1259.1392167224028, 771.32342877765313,
    -176.61502916214059, 12.507343278686905, -0.13857109526572012,
    9.9843695780195716e-6, 1.5056327351493116e-7,
)
_HALF_LOG_2PI = 0.91893853320467274178


def _exp_lgamma(a):
    x = a - 1.0
    z = 0.99999999999980993
    for i, c in enumerate(_LANCZOS):
        z = z + c / (x + (i + 1.0))
    t = x + 7.5
    return jnp.exp(_HALF_LOG_2PI + (x + 0.5) * jnp.log(t) - t + jnp.log(z))


def _dot(a, b):
    return jnp.dot(a, b, preferred_element_type=jnp.float32)


def _fused_kernel(adj_ref, x_ref, w1_ref, w2_ref,
                  phi_ref, lbd_ref, kap_ref,
                  ys_ref, h1s_ref):
    p = pl.program_id(0)
    m = pl.program_id(1)

    @pl.when(jnp.logical_and(p == 0, m == 0))
    def _():
        ys_ref[...] = _dot(x_ref[...], w1_ref[...])

    @pl.when(jnp.logical_and(p == 1, m == 0))
    def _():
        ys_ref[...] = _dot(h1s_ref[...], w2_ref[...])

    z = _dot(adj_ref[...], ys_ref[...])

    @pl.when(p == 0)
    def _():
        h1s_ref[pl.ds(m * _BM, _BM), :] = jax.nn.softplus(z)

    @pl.when(p == 1)
    def _():
        sp = jax.nn.softplus(z)
        lbd = sp[:, :64]
        kap = sp[:, 64:65] + 0.1
        phi = lbd * _exp_lgamma(1.0 + 1.0 / kap)
        phi_ref[...] = phi
        lbd_ref[...] = lbd
        kap_ref[...] = kap


def kernel(adj, x, W1, W2):
    W2p = jnp.pad(W2, ((0, 0), (0, _D - W2.shape[1])))
    phi, lbd, kap = pl.pallas_call(
        _fused_kernel,
        grid=(2, _NM),
        in_specs=[
            pl.BlockSpec((_BM, _N), lambda p, m: (m, 0)),
            pl.BlockSpec((_N, _D), lambda p, m: (0, 0)),
            pl.BlockSpec((_D, _D), lambda p, m: (0, 0)),
            pl.BlockSpec((_D, _D), lambda p, m: (0, 0)),
        ],
        out_specs=[
            pl.BlockSpec((_BM, 64), lambda p, m: (p * m, 0)),
            pl.BlockSpec((_BM, 64), lambda p, m: (p * m, 0)),
            pl.BlockSpec((_BM, 1), lambda p, m: (p * m, 0)),
        ],
        out_shape=[
            jax.ShapeDtypeStruct((_N, 64), jnp.float32),
            jax.ShapeDtypeStruct((_N, 64), jnp.float32),
            jax.ShapeDtypeStruct((_N, 1), jnp.float32),
        ],
        scratch_shapes=[
            pltpu.VMEM((_N, _D), jnp.float32),
            pltpu.VMEM((_N, _D), jnp.float32),
        ],
        compiler_params=pltpu.CompilerParams(
            dimension_semantics=("arbitrary", "arbitrary")),
    )(adj, x, W1, W2p)
    return (phi, lbd, kap)


# speedup vs baseline: 1.0401x; 1.0401x over previous
import jax
import jax.numpy as jnp
from jax.experimental import pallas as pl
from jax.experimental.pallas import tpu as pltpu

_N = 10000
_BM = 400
_NM = _N // _BM
_D = 128

_LANCZOS = (
    676.5203681218851, -1259.1392167224028, 771.32342877765313,
    -176.61502916214059, 12.507343278686905, -0.13857109526572012,
    9.9843695780195716e-6, 1.5056327351493116e-7,
)
_HALF_LOG_2PI = 0.91893853320467274178


def _exp_lgamma(a):
    x = a - 1.0
    z = 0.99999999999980993
    for i, c in enumerate(_LANCZOS):
        z = z + c / (x + (i + 1.0))
    t = x + 7.5
    return jnp.exp(_HALF_LOG_2PI + (x + 0.5) * jnp.log(t) - t + jnp.log(z))


def _dot(a, b):
    return jnp.dot(a, b, preferred_element_type=jnp.float32)


def _fused_kernel(adj_ref, x_ref, w1_ref, w2_ref,
                  phi_ref, lbd_ref, kap_ref,
                  ys_ref, h1s_ref):
    p = pl.program_id(0)
    m = pl.program_id(1)

    @pl.when(jnp.logical_and(p == 0, m == 0))
    def _():
        ys_ref[...] = _dot(x_ref[...], w1_ref[...])

    @pl.when(jnp.logical_and(p == 1, m == 0))
    def _():
        ys_ref[...] = _dot(h1s_ref[...], w2_ref[...])

    z = _dot(adj_ref[...], ys_ref[...])

    @pl.when(p == 0)
    def _():
        h1s_ref[pl.ds(m * _BM, _BM), :] = jax.nn.softplus(z)

    @pl.when(p == 1)
    def _():
        sp = jax.nn.softplus(z)
        lbd = sp[:, :64]
        kap = sp[:, 64:65] + 0.1
        phi = lbd * _exp_lgamma(1.0 + 1.0 / kap)
        phi_ref[...] = phi
        lbd_ref[...] = lbd
        kap_ref[...] = kap


def kernel(adj, x, W1, W2):
    W2p = jnp.pad(W2, ((0, 0), (0, _D - W2.shape[1])))
    phi, lbd, kap = pl.pallas_call(
        _fused_kernel,
        grid=(2, _NM),
        in_specs=[
            pl.BlockSpec((_BM, _N), lambda p, m: (m, 0)),
            pl.BlockSpec((_N, _D), lambda p, m: (0, 0)),
            pl.BlockSpec((_D, _D), lambda p, m: (0, 0)),
            pl.BlockSpec((_D, _D), lambda p, m: (0, 0)),
        ],
        out_specs=[
            pl.BlockSpec((_BM, 64), lambda p, m: (p * m, 0)),
            pl.BlockSpec((_BM, 64), lambda p, m: (p * m, 0)),
            pl.BlockSpec((_BM, 1), lambda p, m: (p * m, 0)),
        ],
        out_shape=[
            jax.ShapeDtypeStruct((_N, 64), jnp.float32),
            jax.ShapeDtypeStruct((_N, 64), jnp.float32),
            jax.ShapeDtypeStruct((_N, 1), jnp.float32),
        ],
        scratch_shapes=[
            pltpu.VMEM((_N, _D), jnp.float32),
            pltpu.VMEM((_N, _D), jnp.float32),
        ],
        compiler_params=pltpu.CompilerParams(
            dimension_semantics=("arbitrary", "arbitrary")),
    )(adj, x, W1, W2p)
    return (phi, lbd, kap)
